# baseline (speedup 1.0000x reference)
import functools

import jax
import jax.numpy as jnp
from jax import lax
from jax.experimental import pallas as pl
from jax.experimental.pallas import tpu as pltpu
from jax.experimental.pallas import tpu_sc as plsc

FEAT_DIM = 80
DEPTH_CHANNELS = 112
X_BOUND = (-50.0, 50.0, 0.5)
Y_BOUND = (-50.0, 50.0, 0.5)
NX = 200
NY = 200
DEPTH_MIN = 1.0
DEPTH_MAX = 57.0

_STAGE = 4
NW = 32
NVOX = NX * NY
ROWS = NVOX // NW
CHUNK = 2048
VECS = CHUNK // 16


def _point_meta(intrinsics, extrinsics, feat_h, feat_w, img_h, img_w):
    D = DEPTH_CHANNELS
    depth_bins = jnp.linspace(DEPTH_MIN, DEPTH_MAX, D)
    ys, xs = jnp.meshgrid(jnp.arange(feat_h, dtype=jnp.float32),
                          jnp.arange(feat_w, dtype=jnp.float32), indexing='ij')
    ds = jnp.broadcast_to(depth_bins[:, None, None], (D, feat_h, feat_w))
    xs = jnp.broadcast_to(xs[None], (D, feat_h, feat_w)) * (img_w / feat_w)
    ys = jnp.broadcast_to(ys[None], (D, feat_h, feat_w)) * (img_h / feat_h)
    frustum = jnp.stack([xs, ys, ds], axis=-1)
    pts = frustum.reshape(-1, 3)
    pts = jnp.stack([pts[:, 0] * pts[:, 2], pts[:, 1] * pts[:, 2], pts[:, 2]], axis=-1)
    inv_K = jnp.linalg.inv(intrinsics)
    cam = jnp.einsum('bnij,pj->bnpi', inv_K, pts)
    ones = jnp.ones_like(cam[..., :1])
    cam_h = jnp.concatenate([cam, ones], axis=-1)
    ego = jnp.einsum('bnij,bnpj->bnpi', extrinsics, cam_h)
    geom = ego[..., :3]
    x_idx = ((geom[..., 0] - X_BOUND[0]) / X_BOUND[2]).astype(jnp.int32)
    y_idx = ((geom[..., 1] - Y_BOUND[0]) / Y_BOUND[2]).astype(jnp.int32)
    valid = (x_idx >= 0) & (x_idx < NX) & (y_idx >= 0) & (y_idx < NY)
    lin = (x_idx * NY + y_idx).reshape(-1)
    valid = valid.reshape(-1)
    P = lin.shape[0]
    hw = feat_h * feat_w
    pidx = jnp.arange(P, dtype=jnp.int32)
    col = (pidx // (D * hw)) * hw + pidx % hw
    owner = jnp.where(valid, lin & (NW - 1), NW)
    row = jnp.where(valid, lin >> 5, 0)
    return (owner << 24) | (row << 13) | col


def _sc_body(meta_hbm, w_hbm, ctx_hbm, out_hbm,
             acc, meta_v, w_v, hit_meta, hit_w, ctxbuf, dma_sem):
    t = lax.axis_index("s") * 2 + lax.axis_index("c")
    zeros16 = jnp.zeros((16,), jnp.float32)

    def zero_body(i, _):
        acc[pl.ds(i * 16, 16)] = zeros16
        return 0
    lax.fori_loop(0, ROWS * FEAT_DIM // 16, zero_body, 0)

    def zero_hits(i, _):
        hit_meta[pl.ds(i * 16, 16)] = jnp.zeros((16,), jnp.int32)
        return 0
    lax.fori_loop(0, (CHUNK + 32) // 16, zero_hits, 0)

    nchunks = meta_hbm.shape[0] // CHUNK

    def chunk_body(ci, _):
        pltpu.sync_copy(meta_hbm.at[pl.ds(ci * CHUNK, CHUNK)], meta_v)
        pltpu.sync_copy(w_hbm.at[pl.ds(ci * CHUNK, CHUNK)], w_v)

        def scan_body(v, nh):
            m = meta_v[pl.ds(v * 16, 16)]
            own = (m >> 24) == t
            if _STAGE == 0:
                return nh + m[0]
            pc = 16 - jnp.sum(jnp.minimum((m >> 24) ^ t, 1))
            if _STAGE == 1:
                return nh + pc

            @pl.when(pc > 0)
            def _():
                w = w_v[pl.ds(v * 16, 16)]
                plsc.store_compressed(hit_meta.at[pl.ds(nh, 16)], m, mask=own)
                plsc.store_compressed(hit_w.at[pl.ds(nh, 16)], w, mask=own)
            return nh + pc

        nh = lax.fori_loop(0, VECS, scan_body, 0)
        ngroups = (nh + 15) >> 4 if _STAGE >= 3 else 0

        def group_body(g, _):
            mv = hit_meta[pl.ds(g * 16, 16)]
            colv = mv & 0x1FFF
            pltpu.async_copy(ctx_hbm.at[colv], ctxbuf, dma_sem).wait()
            cnt = jnp.minimum(nh - g * 16, 16)

            def hit_body(i, _):
                m = hit_meta[pl.ds(g * 16 + i, 16)][0]
                wsc = hit_w[pl.ds(g * 16 + i, 16)][0]
                base = ((m >> 13) & 0x7FF) * FEAT_DIM
                for q in range(FEAT_DIM // 16):
                    plsc.addupdate(acc.at[pl.ds(base + q * 16, 16)],
                                   wsc * ctxbuf[i, pl.ds(q * 16, 16)])
                return 0

            if _STAGE >= 4:
                lax.fori_loop(0, cnt, hit_body, 0)
            return 0

        lax.fori_loop(0, ngroups, group_body, 0)
        return 0

    lax.fori_loop(0, nchunks, chunk_body, 0)
    pltpu.sync_copy(acc, out_hbm.at[t])


def kernel(image_features, depth_dist, context_features, intrinsics, extrinsics, img_h, img_w):
    Bb, Nn, C, Hh, Ww = context_features.shape
    meta = _point_meta(intrinsics, extrinsics, Hh, Ww, img_h, img_w)
    w_flat = depth_dist.reshape(-1)
    ctx = jnp.transpose(context_features, (0, 1, 3, 4, 2)).reshape(Nn * Hh * Ww, C)

    mesh = plsc.VectorSubcoreMesh(core_axis_name="c", subcore_axis_name="s")
    sc = functools.partial(
        pl.kernel, _sc_body, mesh=mesh,
        compiler_params=pltpu.CompilerParams(needs_layout_passes=False, use_tc_tiling_on_sc=False),
        out_type=jax.ShapeDtypeStruct((NW, ROWS * FEAT_DIM), jnp.float32),
        scratch_types=[
            pltpu.VMEM((ROWS * FEAT_DIM,), jnp.float32),
            pltpu.VMEM((CHUNK,), jnp.int32),
            pltpu.VMEM((CHUNK,), jnp.float32),
            pltpu.VMEM((CHUNK + 32,), jnp.int32),
            pltpu.VMEM((CHUNK + 32,), jnp.float32),
            pltpu.VMEM((16, FEAT_DIM), jnp.float32),
            pltpu.SemaphoreType.DMA,
        ],
    )()
    out = sc(meta, w_flat, ctx)

    bev = out.reshape(NW, ROWS, C).transpose(1, 0, 2).reshape(NX, NY, C)
    return jnp.transpose(bev, (2, 0, 1))[None]


# speedup vs baseline: 1.8909x; 1.8909x over previous
import functools

import jax
import jax.numpy as jnp
from jax import lax
from jax.experimental import pallas as pl
from jax.experimental.pallas import tpu as pltpu
from jax.experimental.pallas import tpu_sc as plsc

FEAT_DIM = 80
DEPTH_CHANNELS = 112
X_BOUND = (-50.0, 50.0, 0.5)
Y_BOUND = (-50.0, 50.0, 0.5)
NX = 200
NY = 200
DEPTH_MIN = 1.0
DEPTH_MAX = 57.0

NW = 32
NVOX = NX * NY
ROWS = NVOX // NW
CHUNK = 2048
VECS = CHUNK // 16
UNROLL = 4


def _point_meta(intrinsics, extrinsics, feat_h, feat_w, img_h, img_w):
    D = DEPTH_CHANNELS
    depth_bins = jnp.linspace(DEPTH_MIN, DEPTH_MAX, D)
    ys, xs = jnp.meshgrid(jnp.arange(feat_h, dtype=jnp.float32),
                          jnp.arange(feat_w, dtype=jnp.float32), indexing='ij')
    ds = jnp.broadcast_to(depth_bins[:, None, None], (D, feat_h, feat_w))
    xs = jnp.broadcast_to(xs[None], (D, feat_h, feat_w)) * (img_w / feat_w)
    ys = jnp.broadcast_to(ys[None], (D, feat_h, feat_w)) * (img_h / feat_h)
    frustum = jnp.stack([xs, ys, ds], axis=-1)
    pts = frustum.reshape(-1, 3)
    pts = jnp.stack([pts[:, 0] * pts[:, 2], pts[:, 1] * pts[:, 2], pts[:, 2]], axis=-1)
    inv_K = jnp.linalg.inv(intrinsics)
    cam = jnp.einsum('bnij,pj->bnpi', inv_K, pts)
    ones = jnp.ones_like(cam[..., :1])
    cam_h = jnp.concatenate([cam, ones], axis=-1)
    ego = jnp.einsum('bnij,bnpj->bnpi', extrinsics, cam_h)
    geom = ego[..., :3]
    x_idx = ((geom[..., 0] - X_BOUND[0]) / X_BOUND[2]).astype(jnp.int32)
    y_idx = ((geom[..., 1] - Y_BOUND[0]) / Y_BOUND[2]).astype(jnp.int32)
    valid = (x_idx >= 0) & (x_idx < NX) & (y_idx >= 0) & (y_idx < NY)
    lin = (x_idx * NY + y_idx).reshape(-1)
    valid = valid.reshape(-1)
    P = lin.shape[0]
    hw = feat_h * feat_w
    pidx = jnp.arange(P, dtype=jnp.int32)
    col = (pidx // (D * hw)) * hw + pidx % hw
    owner = jnp.where(valid, lin & (NW - 1), NW)
    row = jnp.where(valid, lin >> 5, 0)
    return (owner << 24) | (row << 13) | col


def _sc_body(meta_hbm, w_hbm, ctx_hbm, out_hbm,
             acc, meta_v, w_v, hit_meta, hit_w, ctxbuf, msem, wsem, gsem):
    t = lax.axis_index("s") * 2 + lax.axis_index("c")
    nchunks = meta_hbm.shape[0] // CHUNK

    def zero_body(i, _):
        acc[pl.ds(i * 16, 16)] = jnp.zeros((16,), jnp.float32)
        return 0
    lax.fori_loop(0, ROWS * FEAT_DIM // 16, zero_body, 0)

    def zero_hits(i, _):
        hit_meta[pl.ds(i * 16, 16)] = jnp.zeros((16,), jnp.int32)
        return 0
    lax.fori_loop(0, (CHUNK + 32) // 16, zero_hits, 0)

    def chunk_copies(ci, b):
        src = meta_hbm.at[pl.ds(ci * CHUNK, CHUNK)]
        cm = pltpu.make_async_copy(src, meta_v.at[pl.ds(b * CHUNK, CHUNK)], msem.at[b])
        srcw = w_hbm.at[pl.ds(ci * CHUNK, CHUNK)]
        cw = pltpu.make_async_copy(srcw, w_v.at[pl.ds(b * CHUNK, CHUNK)], wsem.at[b])
        return cm, cw

    def issue_chunk(ci, b):
        @pl.when(ci < nchunks)
        def _():
            cm, cw = chunk_copies(ci, b)
            cm.start()
            cw.start()

    issue_chunk(0, 0)

    def chunk_body(ci, _):
        b = ci & 1
        issue_chunk(ci + 1, 1 - b)
        cm, cw = chunk_copies(ci, b)
        cm.wait()
        cw.wait()
        mbase = b * CHUNK

        def scan_body(u, nh):
            for k in range(UNROLL):
                off = mbase + (u * UNROLL + k) * 16
                m = meta_v[pl.ds(off, 16)]
                own = (m >> 24) == t
                plsc.store_compressed(hit_meta.at[pl.ds(nh, 16)], m, mask=own)
                w = w_v[pl.ds(off, 16)]
                plsc.store_compressed(hit_w.at[pl.ds(nh, 16)], w, mask=own)
                nh = nh + plsc.all_reduce_population_count(own)[0]
            return nh

        nh = lax.fori_loop(0, VECS // UNROLL, scan_body, 0)
        ngroups = (nh + 15) >> 4

        def gather_copy(g, gb):
            mv = hit_meta[pl.ds(g * 16, 16)]
            colv = mv & 0x1FFF
            return pltpu.make_async_copy(ctx_hbm.at[colv],
                                         ctxbuf.at[pl.ds(gb * 16, 16)], gsem.at[gb])

        def issue_gather(g, gb):
            @pl.when(g < ngroups)
            def _():
                gather_copy(g, gb).start()

        issue_gather(0, 0)

        def group_body(g, _):
            gb = g & 1
            issue_gather(g + 1, 1 - gb)
            gather_copy(g, gb).wait()
            cnt = jnp.minimum(nh - g * 16, 16)
            cbase = gb * 16

            def hit_body(i, _):
                m = hit_meta[pl.ds(g * 16 + i, 16)][0]
                wsc = hit_w[pl.ds(g * 16 + i, 16)][0]
                base = ((m >> 13) & 0x7FF) * FEAT_DIM
                for q in range(FEAT_DIM // 16):
                    plsc.addupdate(acc.at[pl.ds(base + q * 16, 16)],
                                   wsc * ctxbuf[cbase + i, pl.ds(q * 16, 16)])
                return 0

            lax.fori_loop(0, cnt, hit_body, 0)
            return 0

        lax.fori_loop(0, ngroups, group_body, 0)
        return 0

    lax.fori_loop(0, nchunks, chunk_body, 0)
    pltpu.sync_copy(acc, out_hbm.at[t])


def kernel(image_features, depth_dist, context_features, intrinsics, extrinsics, img_h, img_w):
    Bb, Nn, C, Hh, Ww = context_features.shape
    meta = _point_meta(intrinsics, extrinsics, Hh, Ww, img_h, img_w)
    w_flat = depth_dist.reshape(-1)
    ctx = jnp.transpose(context_features, (0, 1, 3, 4, 2)).reshape(Nn * Hh * Ww, C)

    mesh = plsc.VectorSubcoreMesh(core_axis_name="c", subcore_axis_name="s")
    sc = functools.partial(
        pl.kernel, _sc_body, mesh=mesh,
        compiler_params=pltpu.CompilerParams(needs_layout_passes=False,
                                             use_tc_tiling_on_sc=False),
        out_type=jax.ShapeDtypeStruct((NW, ROWS * FEAT_DIM), jnp.float32),
        scratch_types=[
            pltpu.VMEM((ROWS * FEAT_DIM,), jnp.float32),
            pltpu.VMEM((2 * CHUNK,), jnp.int32),
            pltpu.VMEM((2 * CHUNK,), jnp.float32),
            pltpu.VMEM((CHUNK + 32,), jnp.int32),
            pltpu.VMEM((CHUNK + 32,), jnp.float32),
            pltpu.VMEM((32, FEAT_DIM), jnp.float32),
            pltpu.SemaphoreType.DMA((2,)),
            pltpu.SemaphoreType.DMA((2,)),
            pltpu.SemaphoreType.DMA((2,)),
        ],
    )()
    out = sc(meta, w_flat, ctx)

    bev = out.reshape(NW, ROWS, C).transpose(1, 0, 2).reshape(NX, NY, C)
    return jnp.transpose(bev, (2, 0, 1))[None]


# speedup vs baseline: 1.9135x; 1.0119x over previous
import functools

import jax
import jax.numpy as jnp
from jax import lax
from jax.experimental import pallas as pl
from jax.experimental.pallas import tpu as pltpu
from jax.experimental.pallas import tpu_sc as plsc

FEAT_DIM = 80
DEPTH_CHANNELS = 112
X_BOUND = (-50.0, 50.0, 0.5)
Y_BOUND = (-50.0, 50.0, 0.5)
NX = 200
NY = 200
DEPTH_MIN = 1.0
DEPTH_MAX = 57.0

NW = 32
NVOX = NX * NY
ROWS = NVOX // NW
CHUNK = 2048
VECS = CHUNK // 16
UNROLL = 8
GDEPTH = 4


def _point_meta(intrinsics, extrinsics, feat_h, feat_w, img_h, img_w):
    D = DEPTH_CHANNELS
    depth_bins = jnp.linspace(DEPTH_MIN, DEPTH_MAX, D)
    ys, xs = jnp.meshgrid(jnp.arange(feat_h, dtype=jnp.float32),
                          jnp.arange(feat_w, dtype=jnp.float32), indexing='ij')
    ds = jnp.broadcast_to(depth_bins[:, None, None], (D, feat_h, feat_w))
    xs = jnp.broadcast_to(xs[None], (D, feat_h, feat_w)) * (img_w / feat_w)
    ys = jnp.broadcast_to(ys[None], (D, feat_h, feat_w)) * (img_h / feat_h)
    frustum = jnp.stack([xs, ys, ds], axis=-1)
    pts = frustum.reshape(-1, 3)
    pts = jnp.stack([pts[:, 0] * pts[:, 2], pts[:, 1] * pts[:, 2], pts[:, 2]], axis=-1)
    inv_K = jnp.linalg.inv(intrinsics)
    cam = jnp.einsum('bnij,pj->bnpi', inv_K, pts)
    ones = jnp.ones_like(cam[..., :1])
    cam_h = jnp.concatenate([cam, ones], axis=-1)
    ego = jnp.einsum('bnij,bnpj->bnpi', extrinsics, cam_h)
    geom = ego[..., :3]
    x_idx = ((geom[..., 0] - X_BOUND[0]) / X_BOUND[2]).astype(jnp.int32)
    y_idx = ((geom[..., 1] - Y_BOUND[0]) / Y_BOUND[2]).astype(jnp.int32)
    valid = (x_idx >= 0) & (x_idx < NX) & (y_idx >= 0) & (y_idx < NY)
    lin = (x_idx * NY + y_idx).reshape(-1)
    valid = valid.reshape(-1)
    P = lin.shape[0]
    hw = feat_h * feat_w
    pidx = jnp.arange(P, dtype=jnp.int32)
    col = (pidx // (D * hw)) * hw + pidx % hw
    owner = jnp.where(valid, lin & (NW - 1), NW)
    row = jnp.where(valid, lin >> 5, 0)
    return (owner << 24) | (row << 13) | col


def _sc_body(meta_hbm, w_hbm, ctx_hbm, out_hbm,
             acc, meta_v, w_v, hit_meta, hit_w, ctxbuf, msem, wsem, gsem):
    t = lax.axis_index("s") * 2 + lax.axis_index("c")
    nchunks = meta_hbm.shape[0] // CHUNK

    def zero_body(i, _):
        acc[pl.ds(i * 16, 16)] = jnp.zeros((16,), jnp.float32)
        return 0
    lax.fori_loop(0, ROWS * FEAT_DIM // 16, zero_body, 0)

    def zero_hits(i, _):
        hit_meta[pl.ds(i * 16, 16)] = jnp.zeros((16,), jnp.int32)
        return 0
    lax.fori_loop(0, (CHUNK + 32) // 16, zero_hits, 0)

    def chunk_copies(ci, b):
        src = meta_hbm.at[pl.ds(ci * CHUNK, CHUNK)]
        cm = pltpu.make_async_copy(src, meta_v.at[pl.ds(b * CHUNK, CHUNK)], msem.at[b])
        srcw = w_hbm.at[pl.ds(ci * CHUNK, CHUNK)]
        cw = pltpu.make_async_copy(srcw, w_v.at[pl.ds(b * CHUNK, CHUNK)], wsem.at[b])
        return cm, cw

    def issue_chunk(ci, b):
        @pl.when(ci < nchunks)
        def _():
            cm, cw = chunk_copies(ci, b)
            cm.start()
            cw.start()

    issue_chunk(0, 0)

    def chunk_body(ci, _):
        b = ci & 1
        issue_chunk(ci + 1, 1 - b)
        cm, cw = chunk_copies(ci, b)
        cm.wait()
        cw.wait()
        mbase = b * CHUNK

        def scan_body(u, nh):
            for k in range(UNROLL):
                off = mbase + (u * UNROLL + k) * 16
                m = meta_v[pl.ds(off, 16)]
                own = (m >> 24) == t
                plsc.store_compressed(hit_meta.at[pl.ds(nh, 16)], m, mask=own)
                w = w_v[pl.ds(off, 16)]
                plsc.store_compressed(hit_w.at[pl.ds(nh, 16)], w, mask=own)
                nh = nh + plsc.all_reduce_population_count(own)[0]
            return nh

        nh = lax.fori_loop(0, VECS // UNROLL, scan_body, 0)
        ngroups = (nh + 15) >> 4

        def gather_copy(g, gb):
            mv = hit_meta[pl.ds(g * 16, 16)]
            colv = mv & 0x1FFF
            return pltpu.make_async_copy(ctx_hbm.at[colv],
                                         ctxbuf.at[pl.ds(gb * 16, 16)], gsem.at[gb])

        def issue_gather(g, gb):
            @pl.when(g < ngroups)
            def _():
                gather_copy(g, gb).start()

        for pg in range(GDEPTH):
            issue_gather(pg, pg)

        def group_body(g, _):
            gb = g & (GDEPTH - 1)
            gather_copy(g, gb).wait()
            cnt = jnp.minimum(nh - g * 16, 16)
            cbase = gb * 16

            def hit_body(i, _):
                m = hit_meta[pl.ds(g * 16 + i, 16)][0]
                wsc = hit_w[pl.ds(g * 16 + i, 16)][0]
                base = ((m >> 13) & 0x7FF) * FEAT_DIM
                for q in range(FEAT_DIM // 16):
                    plsc.addupdate(acc.at[pl.ds(base + q * 16, 16)],
                                   wsc * ctxbuf[cbase + i, pl.ds(q * 16, 16)])
                return 0

            lax.fori_loop(0, cnt, hit_body, 0)
            issue_gather(g + GDEPTH, gb)
            return 0

        lax.fori_loop(0, ngroups, group_body, 0)
        return 0

    lax.fori_loop(0, nchunks, chunk_body, 0)
    pltpu.sync_copy(acc, out_hbm.at[t])


def kernel(image_features, depth_dist, context_features, intrinsics, extrinsics, img_h, img_w):
    Bb, Nn, C, Hh, Ww = context_features.shape
    meta = _point_meta(intrinsics, extrinsics, Hh, Ww, img_h, img_w)
    w_flat = depth_dist.reshape(-1)
    ctx = jnp.transpose(context_features, (0, 1, 3, 4, 2)).reshape(Nn * Hh * Ww, C)

    mesh = plsc.VectorSubcoreMesh(core_axis_name="c", subcore_axis_name="s")
    sc = functools.partial(
        pl.kernel, _sc_body, mesh=mesh,
        compiler_params=pltpu.CompilerParams(needs_layout_passes=False,
                                             use_tc_tiling_on_sc=False),
        out_type=jax.ShapeDtypeStruct((NW, ROWS * FEAT_DIM), jnp.float32),
        scratch_types=[
            pltpu.VMEM((ROWS * FEAT_DIM,), jnp.float32),
            pltpu.VMEM((2 * CHUNK,), jnp.int32),
            pltpu.VMEM((2 * CHUNK,), jnp.float32),
            pltpu.VMEM((CHUNK + 32,), jnp.int32),
            pltpu.VMEM((CHUNK + 32,), jnp.float32),
            pltpu.VMEM((GDEPTH * 16, FEAT_DIM), jnp.float32),
            pltpu.SemaphoreType.DMA((2,)),
            pltpu.SemaphoreType.DMA((2,)),
            pltpu.SemaphoreType.DMA((GDEPTH,)),
        ],
    )()
    out = sc(meta, w_flat, ctx)

    bev = out.reshape(NW, ROWS, C).transpose(1, 0, 2).reshape(NX, NY, C)
    return jnp.transpose(bev, (2, 0, 1))[None]
